# baseline (speedup 1.0000x reference)
import functools

import jax
import jax.numpy as jnp
from jax import lax
from jax.experimental import pallas as pl
from jax.experimental.pallas import tpu as pltpu
from jax.experimental.pallas import tpu_sc as plsc

E = 64
D = 768
H = 768
DO = 768
N = 8192
BT = 128
G = N // BT + E
NP = G * BT
RB = 256
RG = N // RB


def _routing_body(x_ref, gw_ref, gb_ref, top1_ref, rank_ref, counts_ref, carry_ref):
    i = pl.program_id(0)

    @pl.when(i == 0)
    def _():
        carry_ref[...] = jnp.zeros_like(carry_ref)

    xb = x_ref[...]
    logits = lax.dot_general(xb, gw_ref[...], (((1,), (1,)), ((), ())),
                             preferred_element_type=jnp.float32)
    logits = logits + gb_ref[...]
    iota_e = lax.broadcasted_iota(jnp.int32, (RB, E), 1)
    m = jnp.max(logits, axis=1, keepdims=True)
    eq = logits == m
    idx = jnp.min(jnp.where(eq, iota_e, E), axis=1, keepdims=True)
    onehot = (iota_e == idx).astype(jnp.float32)
    ii = lax.broadcasted_iota(jnp.int32, (RB, RB), 0)
    jj = lax.broadcasted_iota(jnp.int32, (RB, RB), 1)
    tril = (ii > jj).astype(jnp.float32)
    rank_blk = lax.dot_general(tril, onehot, (((1,), (0,)), ((), ())),
                               preferred_element_type=jnp.float32)
    rank_tok = jnp.sum(onehot * (rank_blk + carry_ref[...]), axis=1, keepdims=True)
    carry_ref[...] = carry_ref[...] + jnp.sum(onehot, axis=0, keepdims=True)
    top1_ref[0] = idx
    rank_ref[0] = rank_tok.astype(jnp.int32)
    counts_ref[...] = carry_ref[...].astype(jnp.int32)


def _routing(x_flat, gate_w, gate_b):
    gb = gate_b.reshape(1, E)
    top1, rank, counts = pl.pallas_call(
        _routing_body,
        grid=(RG,),
        in_specs=[
            pl.BlockSpec((RB, D), lambda i: (i, 0)),
            pl.BlockSpec((E, D), lambda i: (0, 0)),
            pl.BlockSpec((1, E), lambda i: (0, 0)),
        ],
        out_specs=[
            pl.BlockSpec((1, RB, 1), lambda i: (i, 0, 0)),
            pl.BlockSpec((1, RB, 1), lambda i: (i, 0, 0)),
            pl.BlockSpec((1, E), lambda i: (0, 0)),
        ],
        out_shape=[
            jax.ShapeDtypeStruct((RG, RB, 1), jnp.int32),
            jax.ShapeDtypeStruct((RG, RB, 1), jnp.int32),
            jax.ShapeDtypeStruct((1, E), jnp.int32),
        ],
        scratch_shapes=[pltpu.VMEM((1, E), jnp.float32)],
    )(x_flat, gate_w, gb)
    return top1.reshape(N), rank.reshape(N), counts.reshape(E)


def _ffn_body(meta_ref, x_ref, w1a_ref, w1b_ref, b1_ref, w2a_ref, w2b_ref,
              b2_ref, y_ref):
    xb = x_ref[...]
    ha = lax.dot_general(xb, w1a_ref[0], (((1,), (1,)), ((), ())),
                         preferred_element_type=jnp.float32)
    hb = lax.dot_general(xb, w1b_ref[0], (((1,), (1,)), ((), ())),
                         preferred_element_type=jnp.float32)
    h = jnp.maximum(jnp.concatenate([ha, hb], axis=1) + b1_ref[0], 0.0)
    ya = lax.dot_general(h[:, : H // 2], w2a_ref[0], (((1,), (1,)), ((), ())),
                         preferred_element_type=jnp.float32)
    yb = lax.dot_general(h[:, H // 2 :], w2b_ref[0], (((1,), (1,)), ((), ())),
                         preferred_element_type=jnp.float32)
    y_ref[...] = ya + yb + b2_ref[0]


def _grouped_ffn(x_sorted, w1, b1, w2, b2, block_meta):
    grid_spec = pltpu.PrefetchScalarGridSpec(
        num_scalar_prefetch=1,
        grid=(G,),
        in_specs=[
            pl.BlockSpec((BT, D), lambda g, m: (m[1, g], 0)),
            pl.BlockSpec((1, H // 2, D), lambda g, m: (m[0, g], 0, 0)),
            pl.BlockSpec((1, H // 2, D), lambda g, m: (m[0, g], 1, 0)),
            pl.BlockSpec((1, 1, H), lambda g, m: (m[0, g], 0, 0)),
            pl.BlockSpec((1, DO, H // 2), lambda g, m: (m[0, g], 0, 0)),
            pl.BlockSpec((1, DO, H // 2), lambda g, m: (m[0, g], 0, 1)),
            pl.BlockSpec((1, 1, DO), lambda g, m: (m[0, g], 0, 0)),
        ],
        out_specs=pl.BlockSpec((BT, DO), lambda g, m: (m[2, g], 0)),
    )
    return pl.pallas_call(
        _ffn_body,
        grid_spec=grid_spec,
        out_shape=jax.ShapeDtypeStruct((NP, DO), jnp.float32),
    )(block_meta, x_sorted, w1, w1, b1.reshape(E, 1, H), w2, w2,
      b2.reshape(E, 1, DO))


NC = 2
NS = 16
L = 16
NW = NC * NS
TPW = N // NW
CH = 64

_sc_params = pltpu.CompilerParams(needs_layout_passes=False)


@functools.cache
def _sc_kernels():
    sc_mesh = plsc.VectorSubcoreMesh(core_axis_name="c", subcore_axis_name="s",
                                     num_cores=NC, num_subcores=NS)

    NCH = TPW // CH

    @functools.partial(
        pl.kernel,
        out_type=[jax.ShapeDtypeStruct((NP, D), jnp.float32),
                  jax.ShapeDtypeStruct((N,), jnp.int32)],
        mesh=sc_mesh,
        compiler_params=_sc_params,
        scratch_types=[
            pltpu.VMEM((TPW,), jnp.int32),
            pltpu.VMEM((TPW,), jnp.int32),
            pltpu.VMEM((E,), jnp.int32),
            pltpu.VMEM((TPW,), jnp.int32),
            pltpu.VMEM((NCH, CH), jnp.int32),
            pltpu.VMEM((CH, D), jnp.float32),
            pltpu.VMEM((CH, D), jnp.float32),
            pltpu.SemaphoreType.DMA,
            pltpu.SemaphoreType.DMA,
        ],
    )
    def sc_dispatch(x_hbm, top1_hbm, rank_hbm, counts_hbm, xs_hbm, pos_hbm,
                    top1_v, rank_v, poffs_v, pos_v, idx_v, rows0_v, rows1_v,
                    sem_in, sem_out):
        wid = lax.axis_index("s") * NC + lax.axis_index("c")
        base = wid * TPW
        rows = (rows0_v, rows1_v)
        c_in0 = pltpu.async_copy(x_hbm.at[pl.ds(base, CH)], rows0_v, sem_in)
        c_in1 = pltpu.async_copy(x_hbm.at[pl.ds(base + CH, CH)], rows1_v, sem_in)
        pltpu.sync_copy(top1_hbm.at[pl.ds(base, TPW)], top1_v)
        pltpu.sync_copy(rank_hbm.at[pl.ds(base, TPW)], rank_v)
        pltpu.sync_copy(counts_hbm, poffs_v)
        carry = jnp.int32(0)
        for k in range(E // L):
            c = poffs_v[pl.ds(k * L, L)]
            p = (c + (BT - 1)) // BT * BT
            poffs_v[pl.ds(k * L, L)] = plsc.cumsum(p) - p + carry
            carry = carry + jnp.sum(p)
        for ch in range(NCH):
            for j in range(CH // L):
                t = top1_v[pl.ds(ch * CH + j * L, L)]
                r = rank_v[pl.ds(ch * CH + j * L, L)]
                pp = plsc.load_gather(poffs_v, [t]) + r
                idx_v[ch, pl.ds(j * L, L)] = pp
                pos_v[pl.ds(ch * CH + j * L, L)] = pp
        pltpu.sync_copy(pos_v, pos_hbm.at[pl.ds(base, TPW)])
        ins = [c_in0, c_in1] + [None] * (NCH - 2)
        outs = [None] * NCH
        for ch in range(NCH):
            b = rows[ch % 2]
            ins[ch].wait()
            outs[ch] = pltpu.async_copy(b, xs_hbm.at[idx_v.at[ch]], sem_out)
            nxt = ch + 2
            if nxt < NCH:
                outs[ch].wait()
                ins[nxt] = pltpu.async_copy(
                    x_hbm.at[pl.ds(base + nxt * CH, CH)], rows[nxt % 2], sem_in)
        for ch in range(max(0, NCH - 2), NCH):
            outs[ch].wait()

    @functools.partial(
        pl.kernel,
        out_type=jax.ShapeDtypeStruct((N, DO), jnp.float32),
        mesh=sc_mesh,
        compiler_params=_sc_params,
        scratch_types=[
            pltpu.VMEM((NCH, CH), jnp.int32),
            pltpu.VMEM((CH, DO), jnp.float32),
            pltpu.VMEM((CH, DO), jnp.float32),
            pltpu.SemaphoreType.DMA,
            pltpu.SemaphoreType.DMA,
        ],
    )
    def sc_combine(y_hbm, pos_hbm, out_hbm, idx_v, rows0_v, rows1_v,
                   sem_in, sem_out):
        wid = lax.axis_index("s") * NC + lax.axis_index("c")
        base = wid * TPW
        rows = (rows0_v, rows1_v)
        for ch in range(NCH):
            pltpu.sync_copy(pos_hbm.at[pl.ds(base + ch * CH, CH)],
                            idx_v.at[ch])
        ins = [None] * NCH
        outs = [None] * NCH
        ins[0] = pltpu.async_copy(y_hbm.at[idx_v.at[0]], rows0_v, sem_in)
        ins[1] = pltpu.async_copy(y_hbm.at[idx_v.at[1]], rows1_v, sem_in)
        for ch in range(NCH):
            b = rows[ch % 2]
            ins[ch].wait()
            outs[ch] = pltpu.async_copy(
                b, out_hbm.at[pl.ds(base + ch * CH, CH)], sem_out)
            nxt = ch + 2
            if nxt < NCH:
                outs[ch].wait()
                ins[nxt] = pltpu.async_copy(
                    y_hbm.at[idx_v.at[nxt]], rows[nxt % 2], sem_in)
        for ch in range(max(0, NCH - 2), NCH):
            outs[ch].wait()

    return sc_dispatch, sc_combine


def kernel(x, gate_w, gate_b, w1, b1, w2, b2):
    Bb, Ss, Dd = x.shape
    x_flat = x.reshape(N, D)

    top1, rank, counts = _routing(x_flat, gate_w, gate_b)

    nblk = (counts + BT - 1) // BT
    used = jnp.sum(nblk)
    g_ids = jnp.arange(G, dtype=jnp.int32)
    active = g_ids < used
    block_expert = jnp.repeat(jnp.arange(E, dtype=jnp.int32), nblk,
                              total_repeat_length=G)
    block_expert = jnp.where(active, block_expert, block_expert[used - 1])
    block_x = jnp.where(active, g_ids, used - 1)
    block_y = block_x
    block_meta = jnp.stack([block_expert, block_x, block_y])

    sc_dispatch, sc_combine = _sc_kernels()
    x_sorted, pos = sc_dispatch(x_flat, top1, rank, counts)

    y_sorted = _grouped_ffn(x_sorted, w1, b1, w2, b2, block_meta)

    out = sc_combine(y_sorted, pos)
    return out.reshape(Bb, Ss, DO)


# speedup vs baseline: 1.0275x; 1.0275x over previous
import functools

import jax
import jax.numpy as jnp
from jax import lax
from jax.experimental import pallas as pl
from jax.experimental.pallas import tpu as pltpu
from jax.experimental.pallas import tpu_sc as plsc

E = 64
D = 768
H = 768
DO = 768
N = 8192
BT = 128
G = N // BT + E
NP = G * BT
RB = 256
RG = N // RB


def _routing_body(x_ref, gw_ref, gb_ref, top1_ref, rank_ref, counts_ref,
                  carry_ref):
    i = pl.program_id(0)

    @pl.when(i == 0)
    def _():
        carry_ref[...] = jnp.zeros_like(carry_ref)

    xb = x_ref[...]
    logits = lax.dot_general(xb, gw_ref[...], (((1,), (1,)), ((), ())),
                             preferred_element_type=jnp.float32)
    logits = logits + gb_ref[...]
    iota_e = lax.broadcasted_iota(jnp.int32, (RB, E), 1)
    m = jnp.max(logits, axis=1, keepdims=True)
    eq = logits == m
    idx = jnp.min(jnp.where(eq, iota_e, E), axis=1, keepdims=True)
    onehot = (iota_e == idx).astype(jnp.float32)
    ii = lax.broadcasted_iota(jnp.int32, (RB, RB), 0)
    jj = lax.broadcasted_iota(jnp.int32, (RB, RB), 1)
    tril = (ii > jj).astype(jnp.float32)
    rank_blk = lax.dot_general(tril, onehot, (((1,), (0,)), ((), ())),
                               preferred_element_type=jnp.float32)
    rank_tok = jnp.sum(onehot * (rank_blk + carry_ref[...]), axis=1, keepdims=True)
    carry_ref[...] = carry_ref[...] + jnp.sum(onehot, axis=0, keepdims=True)
    top1_ref[0] = idx
    rank_ref[0] = rank_tok.astype(jnp.int32)
    counts_ref[...] = carry_ref[...].astype(jnp.int32)


def _routing(x_flat, gate_w, gate_b):
    gb = gate_b.reshape(1, E)
    top1, rank, counts = pl.pallas_call(
        _routing_body,
        grid=(RG,),
        in_specs=[
            pl.BlockSpec((RB, D), lambda i: (i, 0)),
            pl.BlockSpec((E, D), lambda i: (0, 0)),
            pl.BlockSpec((1, E), lambda i: (0, 0)),
        ],
        out_specs=[
            pl.BlockSpec((1, RB, 1), lambda i: (i, 0, 0)),
            pl.BlockSpec((1, RB, 1), lambda i: (i, 0, 0)),
            pl.BlockSpec((1, E), lambda i: (0, 0)),
        ],
        out_shape=[
            jax.ShapeDtypeStruct((RG, RB, 1), jnp.int32),
            jax.ShapeDtypeStruct((RG, RB, 1), jnp.int32),
            jax.ShapeDtypeStruct((1, E), jnp.int32),
        ],
        scratch_shapes=[pltpu.VMEM((1, E), jnp.float32)],
    )(x_flat, gate_w, gb)
    return top1.reshape(N), rank.reshape(N), counts.reshape(E)


def _ffn_body(meta_ref, x_ref, w1_ref, b1_ref, w2_ref, b2_ref, y_ref):
    xb = x_ref[...]
    h = lax.dot_general(xb, w1_ref[0], (((1,), (1,)), ((), ())),
                        preferred_element_type=jnp.float32)
    h = jnp.maximum(h + b1_ref[0], 0.0)
    y = lax.dot_general(h, w2_ref[0], (((1,), (1,)), ((), ())),
                        preferred_element_type=jnp.float32)
    y_ref[...] = y + b2_ref[0]


def _grouped_ffn(x_sorted, w1, b1, w2, b2, block_meta):
    grid_spec = pltpu.PrefetchScalarGridSpec(
        num_scalar_prefetch=1,
        grid=(G,),
        in_specs=[
            pl.BlockSpec((BT, D), lambda g, m: (m[1, g], 0)),
            pl.BlockSpec((1, H, D), lambda g, m: (m[0, g], 0, 0)),
            pl.BlockSpec((1, 1, H), lambda g, m: (m[0, g], 0, 0)),
            pl.BlockSpec((1, DO, H), lambda g, m: (m[0, g], 0, 0)),
            pl.BlockSpec((1, 1, DO), lambda g, m: (m[0, g], 0, 0)),
        ],
        out_specs=pl.BlockSpec((BT, DO), lambda g, m: (m[2, g], 0)),
    )
    return pl.pallas_call(
        _ffn_body,
        grid_spec=grid_spec,
        out_shape=jax.ShapeDtypeStruct((NP, DO), jnp.float32),
    )(block_meta, x_sorted, w1, b1.reshape(E, 1, H), w2, b2.reshape(E, 1, DO))


NC = 2
NS = 16
L = 16
NW = NC * NS
TPW = N // NW
CH = 64

_sc_params = pltpu.CompilerParams(needs_layout_passes=False)


@functools.cache
def _sc_kernels():
    sc_mesh = plsc.VectorSubcoreMesh(core_axis_name="c", subcore_axis_name="s",
                                     num_cores=NC, num_subcores=NS)

    NCH = TPW // CH

    @functools.partial(
        pl.kernel,
        out_type=[jax.ShapeDtypeStruct((NP, D), jnp.float32),
                  jax.ShapeDtypeStruct((N,), jnp.int32)],
        mesh=sc_mesh,
        compiler_params=_sc_params,
        scratch_types=[
            pltpu.VMEM((TPW,), jnp.int32),
            pltpu.VMEM((TPW,), jnp.int32),
            pltpu.VMEM((E,), jnp.int32),
            pltpu.VMEM((TPW,), jnp.int32),
            pltpu.VMEM((NCH, CH), jnp.int32),
            pltpu.VMEM((CH, D), jnp.float32),
            pltpu.VMEM((CH, D), jnp.float32),
            pltpu.SemaphoreType.DMA,
            pltpu.SemaphoreType.DMA,
            pltpu.SemaphoreType.DMA,
            pltpu.SemaphoreType.DMA,
        ],
    )
    def sc_dispatch(x_hbm, top1_hbm, rank_hbm, counts_hbm, xs_hbm, pos_hbm,
                    top1_v, rank_v, poffs_v, pos_v, idx_v, rows0_v, rows1_v,
                    sem_in0, sem_in1, sem_out0, sem_out1):
        wid = lax.axis_index("s") * NC + lax.axis_index("c")
        base = wid * TPW
        rows = (rows0_v, rows1_v)
        sem_in = (sem_in0, sem_in1)
        sem_out = (sem_out0, sem_out1)
        c_in0 = pltpu.async_copy(x_hbm.at[pl.ds(base, CH)], rows0_v, sem_in0)
        c_in1 = pltpu.async_copy(x_hbm.at[pl.ds(base + CH, CH)], rows1_v, sem_in1)
        pltpu.sync_copy(top1_hbm.at[pl.ds(base, TPW)], top1_v)
        pltpu.sync_copy(rank_hbm.at[pl.ds(base, TPW)], rank_v)
        pltpu.sync_copy(counts_hbm, poffs_v)
        carry = jnp.int32(0)
        for k in range(E // L):
            c = poffs_v[pl.ds(k * L, L)]
            p = (c + (BT - 1)) // BT * BT
            poffs_v[pl.ds(k * L, L)] = plsc.cumsum(p) - p + carry
            carry = carry + jnp.sum(p)
        for ch in range(NCH):
            for j in range(CH // L):
                t = top1_v[pl.ds(ch * CH + j * L, L)]
                r = rank_v[pl.ds(ch * CH + j * L, L)]
                pp = plsc.load_gather(poffs_v, [t]) + r
                idx_v[ch, pl.ds(j * L, L)] = pp
                pos_v[pl.ds(ch * CH + j * L, L)] = pp
        pltpu.sync_copy(pos_v, pos_hbm.at[pl.ds(base, TPW)])
        ins = [c_in0, c_in1] + [None] * (NCH - 2)
        outs = [None] * NCH
        for ch in range(NCH):
            b = rows[ch % 2]
            ins[ch].wait()
            outs[ch] = pltpu.async_copy(b, xs_hbm.at[idx_v.at[ch]],
                                        sem_out[ch % 2])
            nxt = ch + 2
            if nxt < NCH:
                outs[ch].wait()
                ins[nxt] = pltpu.async_copy(
                    x_hbm.at[pl.ds(base + nxt * CH, CH)], rows[nxt % 2],
                    sem_in[nxt % 2])
        for ch in range(max(0, NCH - 2), NCH):
            outs[ch].wait()

    @functools.partial(
        pl.kernel,
        out_type=jax.ShapeDtypeStruct((N, DO), jnp.float32),
        mesh=sc_mesh,
        compiler_params=_sc_params,
        scratch_types=[
            pltpu.VMEM((NCH, CH), jnp.int32),
            pltpu.VMEM((CH, DO), jnp.float32),
            pltpu.VMEM((CH, DO), jnp.float32),
            pltpu.SemaphoreType.DMA,
            pltpu.SemaphoreType.DMA,
            pltpu.SemaphoreType.DMA,
            pltpu.SemaphoreType.DMA,
        ],
    )
    def sc_combine(y_hbm, pos_hbm, out_hbm, idx_v, rows0_v, rows1_v,
                   sem_in0, sem_in1, sem_out0, sem_out1):
        wid = lax.axis_index("s") * NC + lax.axis_index("c")
        base = wid * TPW
        rows = (rows0_v, rows1_v)
        sem_in = (sem_in0, sem_in1)
        sem_out = (sem_out0, sem_out1)
        for ch in range(NCH):
            pltpu.sync_copy(pos_hbm.at[pl.ds(base + ch * CH, CH)],
                            idx_v.at[ch])
        ins = [None] * NCH
        outs = [None] * NCH
        ins[0] = pltpu.async_copy(y_hbm.at[idx_v.at[0]], rows0_v, sem_in0)
        ins[1] = pltpu.async_copy(y_hbm.at[idx_v.at[1]], rows1_v, sem_in1)
        for ch in range(NCH):
            b = rows[ch % 2]
            ins[ch].wait()
            outs[ch] = pltpu.async_copy(
                b, out_hbm.at[pl.ds(base + ch * CH, CH)], sem_out[ch % 2])
            nxt = ch + 2
            if nxt < NCH:
                outs[ch].wait()
                ins[nxt] = pltpu.async_copy(
                    y_hbm.at[idx_v.at[nxt]], rows[nxt % 2], sem_in[nxt % 2])
        for ch in range(max(0, NCH - 2), NCH):
            outs[ch].wait()

    return sc_dispatch, sc_combine


def kernel(x, gate_w, gate_b, w1, b1, w2, b2):
    Bb, Ss, Dd = x.shape
    x_flat = x.reshape(N, D)

    top1, rank, counts = _routing(x_flat, gate_w, gate_b)

    nblk = (counts + BT - 1) // BT
    used = jnp.sum(nblk)
    g_ids = jnp.arange(G, dtype=jnp.int32)
    active = g_ids < used
    block_expert = jnp.repeat(jnp.arange(E, dtype=jnp.int32), nblk,
                              total_repeat_length=G)
    block_expert = jnp.where(active, block_expert, block_expert[used - 1])
    block_x = jnp.where(active, g_ids, used - 1)
    block_y = block_x
    block_meta = jnp.stack([block_expert, block_x, block_y])

    sc_dispatch, sc_combine = _sc_kernels()
    x_sorted, pos = sc_dispatch(x_flat, top1, rank, counts)

    y_sorted = _grouped_ffn(x_sorted, w1, b1, w2, b2, block_meta)

    out = sc_combine(y_sorted, pos)
    return out.reshape(Bb, Ss, DO)


# speedup vs baseline: 1.0598x; 1.0315x over previous
import functools

import jax
import jax.numpy as jnp
from jax import lax
from jax.experimental import pallas as pl
from jax.experimental.pallas import tpu as pltpu
from jax.experimental.pallas import tpu_sc as plsc

E = 64
D = 768
H = 768
DO = 768
N = 8192
BT = 128
G = N // BT + E
NP = G * BT
RB = 512
RG = N // RB


def _routing_body(x_ref, gw_ref, gb_ref, top1_ref, rank_ref, counts_ref,
                  carry_ref):
    i = pl.program_id(0)

    @pl.when(i == 0)
    def _():
        carry_ref[...] = jnp.zeros_like(carry_ref)

    xb = x_ref[...]
    logits = lax.dot_general(xb, gw_ref[...], (((1,), (1,)), ((), ())),
                             preferred_element_type=jnp.float32)
    logits = logits + gb_ref[...]
    iota_e = lax.broadcasted_iota(jnp.int32, (RB, E), 1)
    m = jnp.max(logits, axis=1, keepdims=True)
    eq = logits == m
    idx = jnp.min(jnp.where(eq, iota_e, E), axis=1, keepdims=True)
    onehot = (iota_e == idx).astype(jnp.float32)
    ii = lax.broadcasted_iota(jnp.int32, (RB, RB), 0)
    jj = lax.broadcasted_iota(jnp.int32, (RB, RB), 1)
    tril = (ii > jj).astype(jnp.float32)
    rank_blk = lax.dot_general(tril, onehot, (((1,), (0,)), ((), ())),
                               preferred_element_type=jnp.float32)
    rank_tok = jnp.sum(onehot * (rank_blk + carry_ref[...]), axis=1, keepdims=True)
    carry_ref[...] = carry_ref[...] + jnp.sum(onehot, axis=0, keepdims=True)
    top1_ref[0] = idx
    rank_ref[0] = rank_tok.astype(jnp.int32)
    counts_ref[...] = carry_ref[...].astype(jnp.int32)


def _routing(x_flat, gate_w, gate_b):
    gb = gate_b.reshape(1, E)
    top1, rank, counts = pl.pallas_call(
        _routing_body,
        grid=(RG,),
        in_specs=[
            pl.BlockSpec((RB, D), lambda i: (i, 0)),
            pl.BlockSpec((E, D), lambda i: (0, 0)),
            pl.BlockSpec((1, E), lambda i: (0, 0)),
        ],
        out_specs=[
            pl.BlockSpec((1, RB, 1), lambda i: (i, 0, 0)),
            pl.BlockSpec((1, RB, 1), lambda i: (i, 0, 0)),
            pl.BlockSpec((1, E), lambda i: (0, 0)),
        ],
        out_shape=[
            jax.ShapeDtypeStruct((RG, RB, 1), jnp.int32),
            jax.ShapeDtypeStruct((RG, RB, 1), jnp.int32),
            jax.ShapeDtypeStruct((1, E), jnp.int32),
        ],
        scratch_shapes=[pltpu.VMEM((1, E), jnp.float32)],
    )(x_flat, gate_w, gb)
    return top1.reshape(N), rank.reshape(N), counts.reshape(E)


def _ffn_body(meta_ref, x_ref, w1_ref, b1_ref, w2_ref, b2_ref, y_ref):
    xb = x_ref[...]
    h = lax.dot_general(xb, w1_ref[0], (((1,), (1,)), ((), ())),
                        preferred_element_type=jnp.float32)
    h = jnp.maximum(h + b1_ref[0], 0.0)
    y = lax.dot_general(h, w2_ref[0], (((1,), (1,)), ((), ())),
                        preferred_element_type=jnp.float32)
    y_ref[...] = y + b2_ref[0]


def _grouped_ffn(x_sorted, w1, b1, w2, b2, block_meta):
    grid_spec = pltpu.PrefetchScalarGridSpec(
        num_scalar_prefetch=1,
        grid=(G,),
        in_specs=[
            pl.BlockSpec((BT, D), lambda g, m: (m[1, g], 0)),
            pl.BlockSpec((1, H, D), lambda g, m: (m[0, g], 0, 0)),
            pl.BlockSpec((1, 1, H), lambda g, m: (m[0, g], 0, 0)),
            pl.BlockSpec((1, DO, H), lambda g, m: (m[0, g], 0, 0)),
            pl.BlockSpec((1, 1, DO), lambda g, m: (m[0, g], 0, 0)),
        ],
        out_specs=pl.BlockSpec((BT, DO), lambda g, m: (m[2, g], 0)),
    )
    return pl.pallas_call(
        _ffn_body,
        grid_spec=grid_spec,
        out_shape=jax.ShapeDtypeStruct((NP, DO), jnp.float32),
    )(block_meta, x_sorted, w1, b1.reshape(E, 1, H), w2, b2.reshape(E, 1, DO))


NC = 2
NS = 16
L = 16
NW = NC * NS
TPW = N // NW
CH = 64

_sc_params = pltpu.CompilerParams(needs_layout_passes=False)


@functools.cache
def _sc_kernels():
    sc_mesh = plsc.VectorSubcoreMesh(core_axis_name="c", subcore_axis_name="s",
                                     num_cores=NC, num_subcores=NS)

    NCH = TPW // CH

    @functools.partial(
        pl.kernel,
        out_type=[jax.ShapeDtypeStruct((NP, D), jnp.float32),
                  jax.ShapeDtypeStruct((N,), jnp.int32)],
        mesh=sc_mesh,
        compiler_params=_sc_params,
        scratch_types=[
            pltpu.VMEM((TPW,), jnp.int32),
            pltpu.VMEM((TPW,), jnp.int32),
            pltpu.VMEM((E,), jnp.int32),
            pltpu.VMEM((TPW,), jnp.int32),
            pltpu.VMEM((NCH, CH), jnp.int32),
            pltpu.VMEM((CH, D), jnp.float32),
            pltpu.VMEM((CH, D), jnp.float32),
            pltpu.SemaphoreType.DMA,
            pltpu.SemaphoreType.DMA,
            pltpu.SemaphoreType.DMA,
            pltpu.SemaphoreType.DMA,
        ],
    )
    def sc_dispatch(x_hbm, top1_hbm, rank_hbm, counts_hbm, xs_hbm, pos_hbm,
                    top1_v, rank_v, poffs_v, pos_v, idx_v, rows0_v, rows1_v,
                    sem_in0, sem_in1, sem_out0, sem_out1):
        wid = lax.axis_index("s") * NC + lax.axis_index("c")
        base = wid * TPW
        rows = (rows0_v, rows1_v)
        sem_in = (sem_in0, sem_in1)
        sem_out = (sem_out0, sem_out1)
        c_in0 = pltpu.async_copy(x_hbm.at[pl.ds(base, CH)], rows0_v, sem_in0)
        c_in1 = pltpu.async_copy(x_hbm.at[pl.ds(base + CH, CH)], rows1_v, sem_in1)
        pltpu.sync_copy(top1_hbm.at[pl.ds(base, TPW)], top1_v)
        pltpu.sync_copy(rank_hbm.at[pl.ds(base, TPW)], rank_v)
        pltpu.sync_copy(counts_hbm, poffs_v)
        carry = jnp.int32(0)
        for k in range(E // L):
            c = poffs_v[pl.ds(k * L, L)]
            p = (c + (BT - 1)) // BT * BT
            poffs_v[pl.ds(k * L, L)] = plsc.cumsum(p) - p + carry
            carry = carry + jnp.sum(p)
        for ch in range(NCH):
            for j in range(CH // L):
                t = top1_v[pl.ds(ch * CH + j * L, L)]
                r = rank_v[pl.ds(ch * CH + j * L, L)]
                pp = plsc.load_gather(poffs_v, [t]) + r
                idx_v[ch, pl.ds(j * L, L)] = pp
                pos_v[pl.ds(ch * CH + j * L, L)] = pp
        pltpu.sync_copy(pos_v, pos_hbm.at[pl.ds(base, TPW)])
        ins = [c_in0, c_in1] + [None] * (NCH - 2)
        outs = [None] * NCH
        for ch in range(NCH):
            b = rows[ch % 2]
            ins[ch].wait()
            outs[ch] = pltpu.async_copy(b, xs_hbm.at[idx_v.at[ch]],
                                        sem_out[ch % 2])
            nxt = ch + 2
            if nxt < NCH:
                outs[ch].wait()
                ins[nxt] = pltpu.async_copy(
                    x_hbm.at[pl.ds(base + nxt * CH, CH)], rows[nxt % 2],
                    sem_in[nxt % 2])
        for ch in range(max(0, NCH - 2), NCH):
            outs[ch].wait()

    @functools.partial(
        pl.kernel,
        out_type=jax.ShapeDtypeStruct((N, DO), jnp.float32),
        mesh=sc_mesh,
        compiler_params=_sc_params,
        scratch_types=[
            pltpu.VMEM((NCH, CH), jnp.int32),
            pltpu.VMEM((CH, DO), jnp.float32),
            pltpu.VMEM((CH, DO), jnp.float32),
            pltpu.SemaphoreType.DMA,
            pltpu.SemaphoreType.DMA,
            pltpu.SemaphoreType.DMA,
            pltpu.SemaphoreType.DMA,
        ],
    )
    def sc_combine(y_hbm, pos_hbm, out_hbm, idx_v, rows0_v, rows1_v,
                   sem_in0, sem_in1, sem_out0, sem_out1):
        wid = lax.axis_index("s") * NC + lax.axis_index("c")
        base = wid * TPW
        rows = (rows0_v, rows1_v)
        sem_in = (sem_in0, sem_in1)
        sem_out = (sem_out0, sem_out1)
        for ch in range(NCH):
            pltpu.sync_copy(pos_hbm.at[pl.ds(base + ch * CH, CH)],
                            idx_v.at[ch])
        ins = [None] * NCH
        outs = [None] * NCH
        ins[0] = pltpu.async_copy(y_hbm.at[idx_v.at[0]], rows0_v, sem_in0)
        ins[1] = pltpu.async_copy(y_hbm.at[idx_v.at[1]], rows1_v, sem_in1)
        for ch in range(NCH):
            b = rows[ch % 2]
            ins[ch].wait()
            outs[ch] = pltpu.async_copy(
                b, out_hbm.at[pl.ds(base + ch * CH, CH)], sem_out[ch % 2])
            nxt = ch + 2
            if nxt < NCH:
                outs[ch].wait()
                ins[nxt] = pltpu.async_copy(
                    y_hbm.at[idx_v.at[nxt]], rows[nxt % 2], sem_in[nxt % 2])
        for ch in range(max(0, NCH - 2), NCH):
            outs[ch].wait()

    return sc_dispatch, sc_combine


def kernel(x, gate_w, gate_b, w1, b1, w2, b2):
    Bb, Ss, Dd = x.shape
    x_flat = x.reshape(N, D)

    top1, rank, counts = _routing(x_flat, gate_w, gate_b)

    nblk = (counts + BT - 1) // BT
    used = jnp.sum(nblk)
    g_ids = jnp.arange(G, dtype=jnp.int32)
    active = g_ids < used
    block_expert = jnp.repeat(jnp.arange(E, dtype=jnp.int32), nblk,
                              total_repeat_length=G)
    block_expert = jnp.where(active, block_expert, block_expert[used - 1])
    block_x = jnp.where(active, g_ids, used - 1)
    block_y = block_x
    block_meta = jnp.stack([block_expert, block_x, block_y])

    sc_dispatch, sc_combine = _sc_kernels()
    x_sorted, pos = sc_dispatch(x_flat, top1, rank, counts)

    y_sorted = _grouped_ffn(x_sorted, w1, b1, w2, b2, block_meta)

    out = sc_combine(y_sorted, pos)
    return out.reshape(Bb, Ss, DO)
